# baseline (speedup 1.0000x reference)
import functools

import jax
import jax.numpy as jnp
from jax import lax
from jax.experimental import pallas as pl
from jax.experimental.pallas import tpu as pltpu
from jax.experimental.pallas import tpu_sc as plsc

L, D, E, H = 2048, 768, 8, 1024
BT = 128
NT = 39
NTP = 40
P = NT * BT
NW = 32
CHUNK = L // NW
LANES = 16


def _router_body(x_ref, rs_ref, rw_ref, pes_ref,
                 pos0_ref, pos1_ref, w0_ref, w1_ref, te_ref):
    x = x_ref[...]
    var = jnp.mean(x * x, axis=-1, keepdims=True)
    ri = x * lax.rsqrt(var + 1e-6)
    ri = ri * lax.rsqrt(jnp.float32(D)) * rs_ref[...]
    logits = jnp.dot(ri, rw_ref[...], preferred_element_type=jnp.float32)
    probs = jax.nn.softmax(logits, axis=-1)
    iota = lax.broadcasted_iota(jnp.int32, (L, E), 1)
    m0 = jnp.max(logits, axis=-1, keepdims=True)
    i0 = jnp.min(jnp.where(logits == m0, iota, E), axis=-1, keepdims=True)
    l2 = jnp.where(iota == i0, jnp.float32(-jnp.inf), logits)
    m1 = jnp.max(l2, axis=-1, keepdims=True)
    i1 = jnp.min(jnp.where(l2 == m1, iota, E), axis=-1, keepdims=True)
    top2 = (iota == i0) | (iota == i1)
    denom = jnp.sum(jnp.where(top2, probs, 0.0), axis=-1, keepdims=True)
    denom = jnp.where(denom > 0.0, denom, 1.0)
    wall = (probs / denom) * pes_ref[...]
    w0 = jnp.sum(jnp.where(iota == i0, wall, 0.0), axis=-1, keepdims=True)
    w1 = jnp.sum(jnp.where(iota == i1, wall, 0.0), axis=-1, keepdims=True)
    w0_ref[...] = jnp.broadcast_to(w0, (L, LANES))
    w1_ref[...] = jnp.broadcast_to(w1, (L, LANES))

    ind = top2.astype(jnp.float32)
    row = lax.broadcasted_iota(jnp.int32, (L, L), 0)
    col = lax.broadcasted_iota(jnp.int32, (L, L), 1)
    tril = (col <= row).astype(jnp.float32)
    csum = lax.dot_general(tril, ind, (((1,), (0,)), ((), ())),
                           preferred_element_type=jnp.float32)
    counts = lax.slice(csum, (L - 1, 0), (L, E))
    pc = ((counts.astype(jnp.int32) + BT - 1) // BT) * BT
    er = lax.broadcasted_iota(jnp.int32, (E, E), 0)
    ec = lax.broadcasted_iota(jnp.int32, (E, E), 1)
    strict = (er < ec).astype(jnp.float32)
    po = lax.dot_general(pc.astype(jnp.float32), strict, (((1,), (0,)), ((), ())),
                         preferred_element_type=jnp.float32)
    posmat = po + csum - 1.0
    pos0 = jnp.sum(jnp.where(iota == i0, posmat, 0.0), axis=-1, keepdims=True)
    pos1 = jnp.sum(jnp.where(iota == i1, posmat, 0.0), axis=-1, keepdims=True)
    pos0_ref[...] = pos0.astype(jnp.int32)
    pos1_ref[...] = pos1.astype(jnp.int32)

    gend = po + pc.astype(jnp.float32)
    ts = (lax.broadcasted_iota(jnp.int32, (NTP, 1), 0) * BT).astype(jnp.float32)
    te = jnp.sum((ts >= gend).astype(jnp.int32), axis=-1, keepdims=True)
    te_ref[...] = jnp.minimum(te, E - 1)


@functools.cache
def _make_dispatch():
    mesh = plsc.VectorSubcoreMesh(core_axis_name="c", subcore_axis_name="s")

    @functools.partial(
        pl.kernel,
        mesh=mesh,
        out_type=jax.ShapeDtypeStruct((P, D), jnp.float32),
        scratch_types=[
            pltpu.VMEM((CHUNK,), jnp.int32),
            pltpu.VMEM((CHUNK,), jnp.int32),
            pltpu.VMEM((CHUNK, D), jnp.float32),
            pltpu.SemaphoreType.DMA,
            pltpu.SemaphoreType.DMA,
        ],
    )
    def _dispatch(x_hbm, p0_hbm, p1_hbm, xs_hbm, idx0_v, idx1_v, rows_v, s0, s1):
        wid = lax.axis_index("s") * 2 + lax.axis_index("c")
        base = wid * CHUNK
        pltpu.sync_copy(p0_hbm.at[pl.ds(base, CHUNK)], idx0_v)
        pltpu.sync_copy(p1_hbm.at[pl.ds(base, CHUNK)], idx1_v)
        pltpu.sync_copy(x_hbm.at[pl.ds(base, CHUNK)], rows_v)
        c0 = pltpu.async_copy(rows_v, xs_hbm.at[idx0_v], s0)
        c1 = pltpu.async_copy(rows_v, xs_hbm.at[idx1_v], s1)
        c0.wait()
        c1.wait()

    return _dispatch


def _ffn_body(te_ref, xs_ref, gw_ref, lw_ref, ys_ref):
    xt = xs_ref[...]
    g = gw_ref[0]
    g0 = lax.dot_general(xt, g[0], (((1,), (1,)), ((), ())),
                         preferred_element_type=jnp.float32)
    g1 = lax.dot_general(xt, g[1], (((1,), (1,)), ((), ())),
                         preferred_element_type=jnp.float32)
    act = jax.nn.gelu(g0) * g1
    ys_ref[...] = lax.dot_general(act, lw_ref[0], (((1,), (0,)), ((), ())),
                                  preferred_element_type=jnp.float32)


@functools.cache
def _make_combine():
    mesh = plsc.VectorSubcoreMesh(core_axis_name="c", subcore_axis_name="s")

    @functools.partial(
        pl.kernel,
        mesh=mesh,
        out_type=jax.ShapeDtypeStruct((L, D), jnp.float32),
        scratch_types=[
            pltpu.VMEM((CHUNK,), jnp.int32),
            pltpu.VMEM((CHUNK,), jnp.int32),
            pltpu.VMEM((CHUNK, LANES), jnp.float32),
            pltpu.VMEM((CHUNK, LANES), jnp.float32),
            pltpu.VMEM((CHUNK, D), jnp.float32),
            pltpu.VMEM((CHUNK, D), jnp.float32),
            pltpu.SemaphoreType.DMA,
            pltpu.SemaphoreType.DMA,
        ],
    )
    def _combine(ys_hbm, p0_hbm, p1_hbm, w0_hbm, w1_hbm, out_hbm,
                 idx0_v, idx1_v, w0_v, w1_v, y0_v, y1_v, s0, s1):
        wid = lax.axis_index("s") * 2 + lax.axis_index("c")
        base = wid * CHUNK
        pltpu.sync_copy(p0_hbm.at[pl.ds(base, CHUNK)], idx0_v)
        pltpu.sync_copy(p1_hbm.at[pl.ds(base, CHUNK)], idx1_v)
        pltpu.sync_copy(w0_hbm.at[pl.ds(base, CHUNK)], w0_v)
        pltpu.sync_copy(w1_hbm.at[pl.ds(base, CHUNK)], w1_v)
        c0 = pltpu.async_copy(ys_hbm.at[idx0_v], y0_v, s0)
        c1 = pltpu.async_copy(ys_hbm.at[idx1_v], y1_v, s1)
        c0.wait()
        c1.wait()

        def row(i, carry):
            wv0 = w0_v[i]
            wv1 = w1_v[i]
            for j in range(D // LANES):
                sl = pl.ds(j * LANES, LANES)
                y0_v[i, sl] = wv0 * y0_v[i, sl] + wv1 * y1_v[i, sl]
            return carry

        lax.fori_loop(0, CHUNK, row, 0)
        pltpu.sync_copy(y0_v, out_hbm.at[pl.ds(base, CHUNK)])

    return _combine


@jax.jit
def kernel(x, router_scale, router_w, gating_w, linear_w, per_expert_scale):
    x2 = x.reshape(L, D)
    rs = router_scale.reshape(1, D)
    pes = per_expert_scale.reshape(1, E)
    pos0, pos1, w0, w1, te = pl.pallas_call(
        _router_body,
        out_shape=(
            jax.ShapeDtypeStruct((L, 1), jnp.int32),
            jax.ShapeDtypeStruct((L, 1), jnp.int32),
            jax.ShapeDtypeStruct((L, LANES), jnp.float32),
            jax.ShapeDtypeStruct((L, LANES), jnp.float32),
            jax.ShapeDtypeStruct((NTP, 1), jnp.int32),
        ),
    )(x2, rs, router_w, pes)
    p0 = pos0.reshape(L)
    p1 = pos1.reshape(L)
    te1 = te.reshape(NTP)

    xs = _make_dispatch()(x2, p0, p1)

    ys = pl.pallas_call(
        _ffn_body,
        grid_spec=pltpu.PrefetchScalarGridSpec(
            num_scalar_prefetch=1,
            grid=(NT,),
            in_specs=[
                pl.BlockSpec((BT, D), lambda k, te_r: (k, 0)),
                pl.BlockSpec((1, 2, H, D), lambda k, te_r: (te_r[k], 0, 0, 0)),
                pl.BlockSpec((1, H, D), lambda k, te_r: (te_r[k], 0, 0)),
            ],
            out_specs=pl.BlockSpec((BT, D), lambda k, te_r: (k, 0)),
        ),
        out_shape=jax.ShapeDtypeStruct((P, D), jnp.float32),
    )(te1, xs, gating_w, linear_w)

    out = _make_combine()(ys, p0, p1, w0, w1)
    return out.reshape(1, L, D)
